# Initial kernel scaffold; baseline (speedup 1.0000x reference)
import jax
import jax.numpy as jnp
from jax.experimental import pallas as pl


def kernel(node_feature, nf_init, ef_init, edge_index, e_type, node_type, rel_W1, rel_b1, rel_W2, rel_b2, att_W1, att_b1, att_W2, att_b2, upd_W1, upd_b1, upd_W2, upd_b2):
    raise NotImplementedError("write your pallas kernel here")


# speedup vs baseline: 2.3722x; 2.3722x over previous
import functools

import jax
import jax.numpy as jnp
from jax import lax
from jax.experimental import pallas as pl
from jax.experimental.pallas import tpu as pltpu
from jax.experimental.pallas import tpu_sc as plsc

N = 10000
E = 320000
D = 128
DE = 16
T = 3
NT = 2
H = 128
OUT = 128

NC = 2
NS = 16
NW = NC * NS

EW = E // NW
GCH = 64
NGCH = (EW + GCH - 1) // GCH
GTAIL = EW - (NGCH - 1) * GCH
DTAB = 10240
DSTRIDE = DTAB // NS
DZC = 8

NHALF = N // NC
MROWS = NHALF * T
MDUM = 96
MTAB = 15104
MSTRIDE = MTAB // NS
ES = E // NS
SCH = 32
NSCH = ES // SCH
MZC = MSTRIDE // SCH
MZT = MSTRIDE - MZC * SCH


def _idx_body(src2, dst2, et2, rowsa, rowsb, rm0, rm1):
    srcv = src2[...]
    dstv = dst2[...]
    etv = et2[...]
    rowsa[...] = etv * N + srcv
    rowsb[...] = etv * N + dstv
    spread = lax.broadcasted_iota(jnp.int32, srcv.shape, 0) % MDUM
    dummy = MROWS + spread
    in0 = dstv < NHALF
    rm0[...] = jnp.where(in0, dstv * T + etv, dummy)
    rm1[...] = jnp.where(in0, dummy, (dstv - NHALF) * T + etv)


def _build_indices(src, dst, et):
    BE = 4000
    grid = (E // BE,)
    spec = pl.BlockSpec((BE, 1), lambda i: (i, 0))
    outs = pl.pallas_call(
        _idx_body,
        grid=grid,
        in_specs=[spec, spec, spec],
        out_specs=[spec, spec, spec, spec],
        out_shape=[jax.ShapeDtypeStruct((E, 1), jnp.int32)] * 4,
    )(src.reshape(E, 1), dst.reshape(E, 1), et.reshape(E, 1))
    return tuple(o.reshape(E) for o in outs)


def _pre_body(nfa, nfi, wsrc, wdst, atta, attb, ga, gb):
    nf = jnp.concatenate([nfa[...], nfi[...]], axis=1)
    att_a = jnp.dot(nfi[...], atta[...], preferred_element_type=jnp.float32)
    att_b = jnp.dot(nfi[...], attb[...], preferred_element_type=jnp.float32)
    for t in range(T):
        ga[t, :, 0:H] = jnp.dot(nf, wsrc[t], preferred_element_type=jnp.float32)
        ga[t, :, H:2 * H] = att_a
        gb[t, :, 0:H] = jnp.dot(nf, wdst[t], preferred_element_type=jnp.float32)
        gb[t, :, H:2 * H] = att_b


def _precompute(nfa, nfi, wsrc, wdst, atta, attb):
    BN = 2000
    grid = (N // BN,)
    full = lambda s: pl.BlockSpec(s, lambda i: tuple(0 for _ in s))
    return pl.pallas_call(
        _pre_body,
        grid=grid,
        in_specs=[
            pl.BlockSpec((BN, D), lambda i: (i, 0)),
            pl.BlockSpec((BN, D), lambda i: (i, 0)),
            full((T, 2 * D, H)),
            full((T, 2 * D, H)),
            full((D, H)),
            full((D, H)),
        ],
        out_specs=[
            pl.BlockSpec((T, BN, 2 * H), lambda i: (0, i, 0)),
            pl.BlockSpec((T, BN, 2 * H), lambda i: (0, i, 0)),
        ],
        out_shape=[
            jax.ShapeDtypeStruct((T, N, 2 * H), jnp.float32),
            jax.ShapeDtypeStruct((T, N, 2 * H), jnp.float32),
        ],
    )(nfa, nfi, wsrc, wdst, atta, attb)


def _gather_body(ga_hbm, gb_hbm, rowsa_hbm, rowsb_hbm, dst_hbm, zd_hbm,
                 ones_hbm, outa_hbm, outb_hbm, deg_out,
                 idxa, idxb, dstb, bufa, bufb, onesb, dst16, zbuf, degtab,
                 sema, semb):
    c = lax.axis_index("c")
    s = lax.axis_index("s")
    wid = s * NC + c
    base = wid * EW

    pltpu.sync_copy(zd_hbm, zbuf)
    for k in range(DSTRIDE // DZC):
        pltpu.sync_copy(zbuf, degtab.at[pl.ds(s * DSTRIDE + k * DZC, DZC)])
    pltpu.sync_copy(ones_hbm, onesb)
    plsc.subcore_barrier()

    def do_chunk(off, with_deg):
        pltpu.sync_copy(rowsa_hbm.at[pl.ds(base + off, GCH)], idxa)
        pltpu.sync_copy(rowsb_hbm.at[pl.ds(base + off, GCH)], idxb)
        cpa = pltpu.async_copy(ga_hbm.at[idxa], bufa, sema)
        cpb = pltpu.async_copy(gb_hbm.at[idxb], bufb, semb)
        if with_deg:
            pltpu.sync_copy(dst_hbm.at[pl.ds(base + off, GCH)], dstb)
            pltpu.sync_copy(onesb, degtab.at[dstb], add=True)
        cpa.wait()
        cpb.wait()
        pltpu.sync_copy(bufa, outa_hbm.at[pl.ds(base + off, GCH)])
        pltpu.sync_copy(bufb, outb_hbm.at[pl.ds(base + off, GCH)])

    def chunk(j, carry):
        do_chunk(j * GCH, True)
        return carry

    lax.fori_loop(0, NGCH - 1, chunk, 0)
    do_chunk(EW - GCH, False)
    pltpu.sync_copy(dst_hbm.at[pl.ds(base + EW - GTAIL, GTAIL)], dst16)
    pltpu.sync_copy(onesb.at[pl.ds(0, GTAIL)], degtab.at[dst16], add=True)

    plsc.subcore_barrier()
    for k in range(DSTRIDE // DZC):
        pltpu.sync_copy(degtab.at[pl.ds(s * DSTRIDE + k * DZC, DZC)], zbuf)
        pltpu.sync_copy(zbuf, deg_out.at[c, pl.ds(s * DSTRIDE + k * DZC, DZC)])


def _gather(ga2, gb2, rowsa, rowsb, dst):
    zd = jnp.zeros((DZC, H), jnp.float32)
    ones = jnp.ones((GCH, H), jnp.float32)
    mesh = plsc.VectorSubcoreMesh(core_axis_name="c", subcore_axis_name="s")
    fn = functools.partial(
        pl.kernel,
        mesh=mesh,
        out_type=[
            jax.ShapeDtypeStruct((E, 2 * H), jnp.float32),
            jax.ShapeDtypeStruct((E, 2 * H), jnp.float32),
            jax.ShapeDtypeStruct((NC, DTAB, H), jnp.float32),
        ],
        scratch_types=[
            pltpu.VMEM((GCH,), jnp.int32),
            pltpu.VMEM((GCH,), jnp.int32),
            pltpu.VMEM((GCH,), jnp.int32),
            pltpu.VMEM((GCH, 2 * H), jnp.float32),
            pltpu.VMEM((GCH, 2 * H), jnp.float32),
            pltpu.VMEM((GCH, H), jnp.float32),
            pltpu.VMEM((GTAIL,), jnp.int32),
            pltpu.VMEM((DZC, H), jnp.float32),
            pltpu.VMEM_SHARED((DTAB, H), jnp.float32),
            pltpu.SemaphoreType.DMA,
            pltpu.SemaphoreType.DMA,
        ],
    )(_gather_body)
    return fn(ga2, gb2, rowsa, rowsb, dst, zd, ones)


def _edge_body(ga, gb, ef, et3, wef, rb1, w2, rb2, attef, ab1, aw2, ab2, m):
    gav = ga[...]
    gbv = gb[...]
    efv = ef[...]
    et = et3[...]
    hpre = gav[:, 0:H] + gbv[:, 0:H]
    apre = gav[:, H:2 * H] + gbv[:, H:2 * H]
    ah = jax.nn.relu(
        apre + jnp.dot(efv, attef[...], preferred_element_type=jnp.float32)
        + ab1[0, :])
    logit = jnp.dot(ah, aw2[...], preferred_element_type=jnp.float32) + ab2[0, 0]
    att = jax.nn.sigmoid(logit)
    msum = jnp.zeros((ga.shape[0], OUT), jnp.float32)
    for t in range(T):
        h = jax.nn.relu(
            hpre + jnp.dot(efv, wef[t], preferred_element_type=jnp.float32)
            + rb1[t, :])
        mt = jax.nn.relu(
            jnp.dot(h, w2[t], preferred_element_type=jnp.float32) + rb2[t, :])
        msum = msum + jnp.where(et == t, mt, 0.0)
    m[...] = att * msum


def _edge_mlp(gA, gB, ef, et3, wef, rb1, w2, rb2, attef, ab1, aw2, ab2):
    BE = 2000
    grid = (E // BE,)
    full = lambda s: pl.BlockSpec(s, lambda i: tuple(0 for _ in s))
    return pl.pallas_call(
        _edge_body,
        grid=grid,
        in_specs=[
            pl.BlockSpec((BE, 2 * H), lambda i: (i, 0)),
            pl.BlockSpec((BE, 2 * H), lambda i: (i, 0)),
            pl.BlockSpec((BE, DE), lambda i: (i, 0)),
            pl.BlockSpec((BE, 1), lambda i: (i, 0)),
            full((T, DE, H)),
            full((T, H)),
            full((T, H, OUT)),
            full((T, OUT)),
            full((DE, H)),
            full((1, H)),
            full((H, 1)),
            full((1, 1)),
        ],
        out_specs=pl.BlockSpec((BE, OUT), lambda i: (i, 0)),
        out_shape=jax.ShapeDtypeStruct((E, OUT), jnp.float32),
    )(gA, gB, ef, et3, wef, rb1, w2, rb2, attef, ab1, aw2, ab2)


def _scatter_body(m_hbm, rowsm_hbm, zm_hbm, msg_out,
                  msgtab, idxm, mbuf):
    c = lax.axis_index("c")
    s = lax.axis_index("s")
    ebase = s * ES
    ibase = c * E + ebase

    pltpu.sync_copy(zm_hbm, mbuf)
    for k in range(MZC):
        pltpu.sync_copy(mbuf, msgtab.at[pl.ds(s * MSTRIDE + k * SCH, SCH)])
    if MZT:
        pltpu.sync_copy(mbuf.at[pl.ds(0, MZT)],
                        msgtab.at[pl.ds(s * MSTRIDE + MZC * SCH, MZT)])
    plsc.subcore_barrier()

    def chunk(j, carry):
        off = j * SCH
        pltpu.sync_copy(rowsm_hbm.at[pl.ds(ibase + off, SCH)], idxm)
        pltpu.sync_copy(m_hbm.at[pl.ds(ebase + off, SCH)], mbuf)
        pltpu.sync_copy(mbuf, msgtab.at[idxm], add=True)
        return carry

    lax.fori_loop(0, NSCH, chunk, 0)

    plsc.subcore_barrier()
    for k in range(MZC):
        pltpu.sync_copy(msgtab.at[pl.ds(s * MSTRIDE + k * SCH, SCH)], mbuf)
        pltpu.sync_copy(mbuf, msg_out.at[c, pl.ds(s * MSTRIDE + k * SCH, SCH)])
    if MZT:
        pltpu.sync_copy(msgtab.at[pl.ds(s * MSTRIDE + MZC * SCH, MZT)],
                        mbuf.at[pl.ds(0, MZT)])
        pltpu.sync_copy(mbuf.at[pl.ds(0, MZT)],
                        msg_out.at[c, pl.ds(s * MSTRIDE + MZC * SCH, MZT)])


def _scatter(m, rowsm_cat):
    zm = jnp.zeros((SCH, OUT), jnp.float32)
    mesh = plsc.VectorSubcoreMesh(core_axis_name="c", subcore_axis_name="s")
    fn = functools.partial(
        pl.kernel,
        mesh=mesh,
        out_type=jax.ShapeDtypeStruct((NC, MTAB, OUT), jnp.float32),
        scratch_types=[
            pltpu.VMEM_SHARED((MTAB, OUT), jnp.float32),
            pltpu.VMEM((SCH,), jnp.int32),
            pltpu.VMEM((SCH, OUT), jnp.float32),
        ],
    )(_scatter_body)
    return fn(m, rowsm_cat, zm)


def _node_body(nfa, nfi, msg, deg, nt3, w1, b1, w2, b2, out):
    nf = jnp.concatenate([nfa[...], nfi[...]], axis=1)
    aggr = jnp.concatenate([jax.nn.relu(nf), msg[...]], axis=1)
    degsum = deg[0, :, 0:1] + deg[1, :, 0:1]
    aggr = jnp.where(degsum > 0.0, aggr, 0.0)
    nt = nt3[...]
    outs = []
    for k in range(NT):
        h = jax.nn.relu(
            jnp.dot(aggr, w1[k], preferred_element_type=jnp.float32) + b1[k, :])
        outs.append(jax.nn.relu(
            jnp.dot(h, w2[k], preferred_element_type=jnp.float32) + b2[k, :]))
    out[...] = jnp.where(nt == 0, outs[0], outs[1])


def _node_update(nfa, nfi, msg, deg2, nt3, w1, b1, w2, b2):
    BN = 2000
    grid = (N // BN,)
    AGG = 2 * D + T * OUT
    full = lambda s: pl.BlockSpec(s, lambda i: tuple(0 for _ in s))
    return pl.pallas_call(
        _node_body,
        grid=grid,
        in_specs=[
            pl.BlockSpec((BN, D), lambda i: (i, 0)),
            pl.BlockSpec((BN, D), lambda i: (i, 0)),
            pl.BlockSpec((BN, T * OUT), lambda i: (i, 0)),
            pl.BlockSpec((NC, BN, H), lambda i: (0, i, 0)),
            pl.BlockSpec((BN, 1), lambda i: (i, 0)),
            full((NT, AGG, H)),
            full((NT, H)),
            full((NT, H, OUT)),
            full((NT, OUT)),
        ],
        out_specs=pl.BlockSpec((BN, OUT), lambda i: (i, 0)),
        out_shape=jax.ShapeDtypeStruct((N, OUT), jnp.float32),
    )(nfa, nfi, msg, deg2, nt3, w1, b1, w2, b2)


def kernel(node_feature, nf_init, ef_init, edge_index, e_type, node_type,
           rel_W1, rel_b1, rel_W2, rel_b2,
           att_W1, att_b1, att_W2, att_b2,
           upd_W1, upd_b1, upd_W2, upd_b2):
    src = edge_index[0]
    dst = edge_index[1]

    wsrc = rel_W1[:, :2 * D, :]
    wdst = rel_W1[:, 2 * D:4 * D, :]
    wef = rel_W1[:, 4 * D:, :]
    atta = att_W1[:D, :]
    attb = att_W1[D:2 * D, :]
    attef = att_W1[2 * D:, :]

    rowsa, rowsb, rm0, rm1 = _build_indices(src, dst, e_type)
    GA, GB = _precompute(node_feature, nf_init, wsrc, wdst, atta, attb)
    gA, gB, deg_raw = _gather(GA.reshape(T * N, 2 * H),
                              GB.reshape(T * N, 2 * H), rowsa, rowsb, dst)

    et3 = e_type.reshape(E, 1)
    m = _edge_mlp(gA, gB, ef_init, et3, wef, rel_b1, rel_W2, rel_b2,
                  attef, att_b1.reshape(1, H), att_W2, att_b2.reshape(1, 1))

    msg_raw = _scatter(m, jnp.concatenate([rm0, rm1]))
    msg = msg_raw[:, :MROWS, :].reshape(NC, NHALF, T * OUT).reshape(N, T * OUT)
    deg2 = deg_raw[:, :N, :]

    nt3 = node_type.reshape(N, 1)
    return _node_update(node_feature, nf_init, msg, deg2, nt3,
                        upd_W1, upd_b1, upd_W2, upd_b2)


# speedup vs baseline: 2.6987x; 1.1376x over previous
import functools

import jax
import jax.numpy as jnp
from jax import lax
from jax.experimental import pallas as pl
from jax.experimental.pallas import tpu as pltpu
from jax.experimental.pallas import tpu_sc as plsc

N = 10000
E = 320000
D = 128
DE = 16
T = 3
NT = 2
H = 128
OUT = 128

NC = 2
NS = 16
NW = NC * NS

EW = E // NW
GCH = 64
NGCH = (EW + GCH - 1) // GCH
NFULL = NGCH - 1
GTAIL = EW - NFULL * GCH
DTAB = 5632
DSTRIDE = DTAB // NS
DZC = 16
DDUMB = 5504

NHALF = N // NC
MROWS = NHALF * T
MDUM = 32
MTAB = 15032
MSTRIDE = 944
ES = E // NS
SCH = 32
NSCH = ES // SCH


def _idx_body(src2, dst2, et2, rowsa, rowsb, rm0, rm1, rd0, rd1):
    srcv = src2[...]
    dstv = dst2[...]
    etv = et2[...]
    rowsa[...] = etv * N + srcv
    rowsb[...] = etv * N + dstv
    iot = lax.broadcasted_iota(jnp.int32, srcv.shape, 0)
    dum_m = MROWS + iot % MDUM
    dum_d = DDUMB + iot % 128
    in0 = dstv < NHALF
    rm0[...] = jnp.where(in0, dstv * T + etv, dum_m)
    rm1[...] = jnp.where(in0, dum_m, (dstv - NHALF) * T + etv)
    rd0[...] = jnp.where(in0, dstv, dum_d)
    rd1[...] = jnp.where(in0, dum_d, dstv - NHALF)


def _build_indices(src, dst, et):
    BE = 4000
    grid = (E // BE,)
    spec = pl.BlockSpec((BE, 1), lambda i: (i, 0))
    outs = pl.pallas_call(
        _idx_body,
        grid=grid,
        in_specs=[spec, spec, spec],
        out_specs=[spec] * 6,
        out_shape=[jax.ShapeDtypeStruct((E, 1), jnp.int32)] * 6,
    )(src.reshape(E, 1), dst.reshape(E, 1), et.reshape(E, 1))
    return tuple(o.reshape(E) for o in outs)


def _pre_body(nfa, nfi, wsrc, wdst, atta, attb, ga, gb):
    nf = jnp.concatenate([nfa[...], nfi[...]], axis=1)
    att_a = jnp.dot(nfi[...], atta[...], preferred_element_type=jnp.float32)
    att_b = jnp.dot(nfi[...], attb[...], preferred_element_type=jnp.float32)
    for t in range(T):
        ga[t, :, 0:H] = jnp.dot(nf, wsrc[t], preferred_element_type=jnp.float32)
        ga[t, :, H:2 * H] = att_a
        gb[t, :, 0:H] = jnp.dot(nf, wdst[t], preferred_element_type=jnp.float32)
        gb[t, :, H:2 * H] = att_b


def _precompute(nfa, nfi, wsrc, wdst, atta, attb):
    BN = 2000
    grid = (N // BN,)
    full = lambda s: pl.BlockSpec(s, lambda i: tuple(0 for _ in s))
    return pl.pallas_call(
        _pre_body,
        grid=grid,
        in_specs=[
            pl.BlockSpec((BN, D), lambda i: (i, 0)),
            pl.BlockSpec((BN, D), lambda i: (i, 0)),
            full((T, 2 * D, H)),
            full((T, 2 * D, H)),
            full((D, H)),
            full((D, H)),
        ],
        out_specs=[
            pl.BlockSpec((T, BN, 2 * H), lambda i: (0, i, 0)),
            pl.BlockSpec((T, BN, 2 * H), lambda i: (0, i, 0)),
        ],
        out_shape=[
            jax.ShapeDtypeStruct((T, N, 2 * H), jnp.float32),
            jax.ShapeDtypeStruct((T, N, 2 * H), jnp.float32),
        ],
    )(nfa, nfi, wsrc, wdst, atta, attb)


def _gather_body(ga_hbm, gb_hbm, rowsa_hbm, rowsb_hbm, rowsd_hbm, zd_hbm,
                 ones_hbm, outa_hbm, outb_hbm, deg_out,
                 idxa0, idxb0, dstd0, bufa0, bufb0,
                 idxa1, idxb1, dstd1, bufa1, bufb1,
                 onesb, dst16, zbuf, degtab,
                 seml0, seml1, semg0, semg1, semw0, semw1):
    c = lax.axis_index("c")
    s = lax.axis_index("s")
    wid = s * NC + c
    base = wid * EW
    dbase = c * E + base

    pltpu.sync_copy(zd_hbm, zbuf)
    for k in range(DSTRIDE // DZC):
        pltpu.sync_copy(zbuf, degtab.at[pl.ds(s * DSTRIDE + k * DZC, DZC)])
    pltpu.sync_copy(ones_hbm, onesb)
    plsc.subcore_barrier()

    sets = ((idxa0, idxb0, dstd0, bufa0, bufb0, seml0, semg0, semw0),
            (idxa1, idxb1, dstd1, bufa1, bufb1, seml1, semg1, semw1))

    def fire_loads(off, st):
        ia, ib, dd, ba, bb, sl, sg, sw = st
        pltpu.async_copy(rowsa_hbm.at[pl.ds(base + off, GCH)], ia, sl)
        pltpu.async_copy(rowsb_hbm.at[pl.ds(base + off, GCH)], ib, sl)
        pltpu.async_copy(rowsd_hbm.at[pl.ds(dbase + off, GCH)], dd, sl)

    def drain_loads(off, st):
        ia, ib, dd, ba, bb, sl, sg, sw = st
        pltpu.make_async_copy(rowsa_hbm.at[pl.ds(base + off, GCH)], ia, sl).wait()
        pltpu.make_async_copy(rowsb_hbm.at[pl.ds(base + off, GCH)], ib, sl).wait()
        pltpu.make_async_copy(rowsd_hbm.at[pl.ds(dbase + off, GCH)], dd, sl).wait()

    def drain_writes(st):
        ia, ib, dd, ba, bb, sl, sg, sw = st
        pltpu.make_async_copy(ba, outa_hbm.at[pl.ds(base, GCH)], sw).wait()
        pltpu.make_async_copy(bb, outb_hbm.at[pl.ds(base, GCH)], sw).wait()

    def sub_iter(j, st):
        ia, ib, dd, ba, bb, sl, sg, sw = st

        @pl.when(j >= 2)
        def _():
            drain_writes(st)

        drain_loads(j * GCH, st)
        cpa = pltpu.async_copy(ga_hbm.at[ia], ba, sg)
        cpb = pltpu.async_copy(gb_hbm.at[ib], bb, sg)
        pltpu.sync_copy(onesb, degtab.at[dd], add=True)
        cpa.wait()
        cpb.wait()

        @pl.when(j < NFULL - 2)
        def _():
            fire_loads((j + 2) * GCH, st)

        pltpu.async_copy(ba, outa_hbm.at[pl.ds(base + j * GCH, GCH)], sw)
        pltpu.async_copy(bb, outb_hbm.at[pl.ds(base + j * GCH, GCH)], sw)

    fire_loads(0, sets[0])
    fire_loads(GCH, sets[1])

    def pair(i, carry):
        sub_iter(2 * i, sets[0])
        sub_iter(2 * i + 1, sets[1])
        return carry

    lax.fori_loop(0, NFULL // 2, pair, 0)
    drain_writes(sets[0])
    drain_writes(sets[1])

    off = EW - GCH
    pltpu.sync_copy(rowsa_hbm.at[pl.ds(base + off, GCH)], idxa0)
    pltpu.sync_copy(rowsb_hbm.at[pl.ds(base + off, GCH)], idxb0)
    cpa = pltpu.async_copy(ga_hbm.at[idxa0], bufa0, semg0)
    cpb = pltpu.async_copy(gb_hbm.at[idxb0], bufb0, semg0)
    cpa.wait()
    cpb.wait()
    pltpu.sync_copy(bufa0, outa_hbm.at[pl.ds(base + off, GCH)])
    pltpu.sync_copy(bufb0, outb_hbm.at[pl.ds(base + off, GCH)])
    pltpu.sync_copy(rowsd_hbm.at[pl.ds(dbase + EW - GTAIL, GTAIL)], dst16)
    pltpu.sync_copy(onesb.at[pl.ds(0, GTAIL)], degtab.at[dst16], add=True)

    plsc.subcore_barrier()
    for k in range(DSTRIDE // DZC):
        pltpu.sync_copy(degtab.at[pl.ds(s * DSTRIDE + k * DZC, DZC)], zbuf)
        pltpu.sync_copy(zbuf, deg_out.at[c, pl.ds(s * DSTRIDE + k * DZC, DZC)])


def _gather(ga2, gb2, rowsa, rowsb, rowsd_cat):
    zd = jnp.zeros((DZC, H), jnp.float32)
    ones = jnp.ones((GCH, H), jnp.float32)
    mesh = plsc.VectorSubcoreMesh(core_axis_name="c", subcore_axis_name="s")
    fn = functools.partial(
        pl.kernel,
        mesh=mesh,
        out_type=[
            jax.ShapeDtypeStruct((E, 2 * H), jnp.float32),
            jax.ShapeDtypeStruct((E, 2 * H), jnp.float32),
            jax.ShapeDtypeStruct((NC, DTAB, H), jnp.float32),
        ],
        scratch_types=[
            pltpu.VMEM((GCH,), jnp.int32),
            pltpu.VMEM((GCH,), jnp.int32),
            pltpu.VMEM((GCH,), jnp.int32),
            pltpu.VMEM((GCH, 2 * H), jnp.float32),
            pltpu.VMEM((GCH, 2 * H), jnp.float32),
            pltpu.VMEM((GCH,), jnp.int32),
            pltpu.VMEM((GCH,), jnp.int32),
            pltpu.VMEM((GCH,), jnp.int32),
            pltpu.VMEM((GCH, 2 * H), jnp.float32),
            pltpu.VMEM((GCH, 2 * H), jnp.float32),
            pltpu.VMEM((GCH, H), jnp.float32),
            pltpu.VMEM((GTAIL,), jnp.int32),
            pltpu.VMEM((DZC, H), jnp.float32),
            pltpu.VMEM_SHARED((DTAB, H), jnp.float32),
            pltpu.SemaphoreType.DMA,
            pltpu.SemaphoreType.DMA,
            pltpu.SemaphoreType.DMA,
            pltpu.SemaphoreType.DMA,
            pltpu.SemaphoreType.DMA,
            pltpu.SemaphoreType.DMA,
        ],
    )(_gather_body)
    return fn(ga2, gb2, rowsa, rowsb, rowsd_cat, zd, ones)


def _edge_body(ga, gb, ef, et3, wef, rb1, w2, rb2, attef, ab1, aw2, ab2, m):
    gav = ga[...]
    gbv = gb[...]
    efv = ef[...]
    et = et3[...]
    hpre = gav[:, 0:H] + gbv[:, 0:H]
    apre = gav[:, H:2 * H] + gbv[:, H:2 * H]
    ah = jax.nn.relu(
        apre + jnp.dot(efv, attef[...], preferred_element_type=jnp.float32)
        + ab1[0, :])
    logit = jnp.dot(ah, aw2[...], preferred_element_type=jnp.float32) + ab2[0, 0]
    att = jax.nn.sigmoid(logit)
    msum = jnp.zeros((ga.shape[0], OUT), jnp.float32)
    for t in range(T):
        h = jax.nn.relu(
            hpre + jnp.dot(efv, wef[t], preferred_element_type=jnp.float32)
            + rb1[t, :])
        mt = jax.nn.relu(
            jnp.dot(h, w2[t], preferred_element_type=jnp.float32) + rb2[t, :])
        msum = msum + jnp.where(et == t, mt, 0.0)
    m[...] = att * msum


def _edge_mlp(gA, gB, ef, et3, wef, rb1, w2, rb2, attef, ab1, aw2, ab2):
    BE = 2000
    grid = (E // BE,)
    full = lambda s: pl.BlockSpec(s, lambda i: tuple(0 for _ in s))
    return pl.pallas_call(
        _edge_body,
        grid=grid,
        in_specs=[
            pl.BlockSpec((BE, 2 * H), lambda i: (i, 0)),
            pl.BlockSpec((BE, 2 * H), lambda i: (i, 0)),
            pl.BlockSpec((BE, DE), lambda i: (i, 0)),
            pl.BlockSpec((BE, 1), lambda i: (i, 0)),
            full((T, DE, H)),
            full((T, H)),
            full((T, H, OUT)),
            full((T, OUT)),
            full((DE, H)),
            full((1, H)),
            full((H, 1)),
            full((1, 1)),
        ],
        out_specs=pl.BlockSpec((BE, OUT), lambda i: (i, 0)),
        out_shape=jax.ShapeDtypeStruct((E, OUT), jnp.float32),
    )(gA, gB, ef, et3, wef, rb1, w2, rb2, attef, ab1, aw2, ab2)


def _scatter_body(m_hbm, rowsm_hbm, zm_hbm, msg_out,
                  msgtab, idxm0, mbuf0, idxm1, mbuf1,
                  seml0, seml1, sema0, sema1):
    c = lax.axis_index("c")
    s = lax.axis_index("s")
    ebase = s * ES
    ibase = c * E + ebase
    sbase = s * MSTRIDE

    pltpu.sync_copy(zm_hbm, mbuf0)

    @pl.when(s < NS - 1)
    def _():
        for k in range(29):
            pltpu.sync_copy(mbuf0, msgtab.at[pl.ds(sbase + k * SCH, SCH)])
        pltpu.sync_copy(mbuf0.at[pl.ds(0, 16)],
                        msgtab.at[pl.ds(sbase + 29 * SCH, 16)])

    @pl.when(s == NS - 1)
    def _():
        for k in range(27):
            pltpu.sync_copy(mbuf0, msgtab.at[pl.ds(sbase + k * SCH, SCH)])
        pltpu.sync_copy(mbuf0.at[pl.ds(0, 8)],
                        msgtab.at[pl.ds(sbase + 27 * SCH, 8)])

    plsc.subcore_barrier()

    sets = ((idxm0, mbuf0, seml0, sema0), (idxm1, mbuf1, seml1, sema1))

    def fire_loads(off, st):
        ix, mb, sl, sa = st
        pltpu.async_copy(rowsm_hbm.at[pl.ds(ibase + off, SCH)], ix, sl)
        pltpu.async_copy(m_hbm.at[pl.ds(ebase + off, SCH)], mb, sl)

    def drain_loads(off, st):
        ix, mb, sl, sa = st
        pltpu.make_async_copy(rowsm_hbm.at[pl.ds(ibase + off, SCH)], ix, sl).wait()
        pltpu.make_async_copy(m_hbm.at[pl.ds(ebase + off, SCH)], mb, sl).wait()

    def drain_add(st):
        ix, mb, sl, sa = st
        pltpu.make_async_copy(zm_hbm, mb, sa).wait()

    def sub_iter(j, p):
        st = sets[p]
        other = sets[1 - p]
        ix, mb, sl, sa = st
        drain_loads(j * SCH, st)
        pltpu.async_copy(mb, msgtab.at[ix], sa, add=True)

        @pl.when(j >= 1)
        def _():
            drain_add(other)

        @pl.when(j < NSCH - 1)
        def _():
            fire_loads((j + 1) * SCH, other)

    fire_loads(0, sets[0])

    def pair(i, carry):
        sub_iter(2 * i, 0)
        sub_iter(2 * i + 1, 1)
        return carry

    lax.fori_loop(0, NSCH // 2, pair, 0)
    sub_iter(NSCH - 1, 0)
    drain_add(sets[0])

    plsc.subcore_barrier()

    @pl.when(s < NS - 1)
    def _():
        for k in range(29):
            pltpu.sync_copy(msgtab.at[pl.ds(sbase + k * SCH, SCH)], mbuf0)
            pltpu.sync_copy(mbuf0, msg_out.at[c, pl.ds(sbase + k * SCH, SCH)])
        pltpu.sync_copy(msgtab.at[pl.ds(sbase + 29 * SCH, 16)],
                        mbuf0.at[pl.ds(0, 16)])
        pltpu.sync_copy(mbuf0.at[pl.ds(0, 16)],
                        msg_out.at[c, pl.ds(sbase + 29 * SCH, 16)])

    @pl.when(s == NS - 1)
    def _():
        for k in range(27):
            pltpu.sync_copy(msgtab.at[pl.ds(sbase + k * SCH, SCH)], mbuf0)
            pltpu.sync_copy(mbuf0, msg_out.at[c, pl.ds(sbase + k * SCH, SCH)])
        pltpu.sync_copy(msgtab.at[pl.ds(sbase + 27 * SCH, 8)],
                        mbuf0.at[pl.ds(0, 8)])
        pltpu.sync_copy(mbuf0.at[pl.ds(0, 8)],
                        msg_out.at[c, pl.ds(sbase + 27 * SCH, 8)])


def _scatter(m, rowsm_cat):
    zm = jnp.zeros((SCH, OUT), jnp.float32)
    mesh = plsc.VectorSubcoreMesh(core_axis_name="c", subcore_axis_name="s")
    fn = functools.partial(
        pl.kernel,
        mesh=mesh,
        out_type=jax.ShapeDtypeStruct((NC, MTAB, OUT), jnp.float32),
        scratch_types=[
            pltpu.VMEM_SHARED((MTAB, OUT), jnp.float32),
            pltpu.VMEM((SCH,), jnp.int32),
            pltpu.VMEM((SCH, OUT), jnp.float32),
            pltpu.VMEM((SCH,), jnp.int32),
            pltpu.VMEM((SCH, OUT), jnp.float32),
            pltpu.SemaphoreType.DMA,
            pltpu.SemaphoreType.DMA,
            pltpu.SemaphoreType.DMA,
            pltpu.SemaphoreType.DMA,
        ],
    )(_scatter_body)
    return fn(m, rowsm_cat, zm)


def _node_body(nfa, nfi, msg, deg, nt3, w1, b1, w2, b2, out):
    nf = jnp.concatenate([nfa[...], nfi[...]], axis=1)
    aggr = jnp.concatenate([jax.nn.relu(nf), msg[...]], axis=1)
    degsum = deg[:, 0:1]
    aggr = jnp.where(degsum > 0.0, aggr, 0.0)
    nt = nt3[...]
    outs = []
    for k in range(NT):
        h = jax.nn.relu(
            jnp.dot(aggr, w1[k], preferred_element_type=jnp.float32) + b1[k, :])
        outs.append(jax.nn.relu(
            jnp.dot(h, w2[k], preferred_element_type=jnp.float32) + b2[k, :]))
    out[...] = jnp.where(nt == 0, outs[0], outs[1])


def _node_update(nfa, nfi, msg, deg2, nt3, w1, b1, w2, b2):
    BN = 2000
    grid = (N // BN,)
    AGG = 2 * D + T * OUT
    full = lambda s: pl.BlockSpec(s, lambda i: tuple(0 for _ in s))
    return pl.pallas_call(
        _node_body,
        grid=grid,
        in_specs=[
            pl.BlockSpec((BN, D), lambda i: (i, 0)),
            pl.BlockSpec((BN, D), lambda i: (i, 0)),
            pl.BlockSpec((BN, T * OUT), lambda i: (i, 0)),
            pl.BlockSpec((BN, H), lambda i: (i, 0)),
            pl.BlockSpec((BN, 1), lambda i: (i, 0)),
            full((NT, AGG, H)),
            full((NT, H)),
            full((NT, H, OUT)),
            full((NT, OUT)),
        ],
        out_specs=pl.BlockSpec((BN, OUT), lambda i: (i, 0)),
        out_shape=jax.ShapeDtypeStruct((N, OUT), jnp.float32),
    )(nfa, nfi, msg, deg2, nt3, w1, b1, w2, b2)


def kernel(node_feature, nf_init, ef_init, edge_index, e_type, node_type,
           rel_W1, rel_b1, rel_W2, rel_b2,
           att_W1, att_b1, att_W2, att_b2,
           upd_W1, upd_b1, upd_W2, upd_b2):
    src = edge_index[0]
    dst = edge_index[1]

    wsrc = rel_W1[:, :2 * D, :]
    wdst = rel_W1[:, 2 * D:4 * D, :]
    wef = rel_W1[:, 4 * D:, :]
    atta = att_W1[:D, :]
    attb = att_W1[D:2 * D, :]
    attef = att_W1[2 * D:, :]

    rowsa, rowsb, rm0, rm1, rd0, rd1 = _build_indices(src, dst, e_type)
    GA, GB = _precompute(node_feature, nf_init, wsrc, wdst, atta, attb)
    gA, gB, deg_raw = _gather(GA.reshape(T * N, 2 * H),
                              GB.reshape(T * N, 2 * H), rowsa, rowsb,
                              jnp.concatenate([rd0, rd1]))

    et3 = e_type.reshape(E, 1)
    m = _edge_mlp(gA, gB, ef_init, et3, wef, rel_b1, rel_W2, rel_b2,
                  attef, att_b1.reshape(1, H), att_W2, att_b2.reshape(1, 1))

    msg_raw = _scatter(m, jnp.concatenate([rm0, rm1]))
    msg = msg_raw[:, :MROWS, :].reshape(NC, NHALF, T * OUT).reshape(N, T * OUT)
    deg2 = deg_raw[:, :NHALF, :].reshape(N, H)

    nt3 = node_type.reshape(N, 1)
    return _node_update(node_feature, nf_init, msg, deg2, nt3,
                        upd_W1, upd_b1, upd_W2, upd_b2)


# speedup vs baseline: 2.7549x; 1.0208x over previous
import functools

import jax
import jax.numpy as jnp
from jax import lax
from jax.experimental import pallas as pl
from jax.experimental.pallas import tpu as pltpu
from jax.experimental.pallas import tpu_sc as plsc

N = 10000
E = 320000
D = 128
DE = 16
T = 3
NT = 2
H = 128
OUT = 128

NC = 2
NS = 16
NW = NC * NS

EW = E // NW
GCH = 64
NGCH = (EW + GCH - 1) // GCH
NFULL = NGCH - 1
GTAIL = EW - NFULL * GCH
DTAB = 5632
DSTRIDE = DTAB // NS
DZC = 16
DDUMB = 5504

NHALF = N // NC
MROWS = NHALF * T
MDUM = 32
MTAB = 15032
MSTRIDE = 944
ES = E // NS
SCH = 32
NSCH = ES // SCH


def _idx_body(src2, dst2, et2, rowsa, rowsb, rm0, rm1, rd0, rd1):
    srcv = src2[...]
    dstv = dst2[...]
    etv = et2[...]
    rowsa[...] = etv * N + srcv
    rowsb[...] = etv * N + dstv
    iot = lax.broadcasted_iota(jnp.int32, srcv.shape, 0)
    dum_m = MROWS + iot % MDUM
    dum_d = DDUMB + iot % 128
    in0 = dstv < NHALF
    rm0[...] = jnp.where(in0, dstv * T + etv, dum_m)
    rm1[...] = jnp.where(in0, dum_m, (dstv - NHALF) * T + etv)
    rd0[...] = jnp.where(in0, dstv, dum_d)
    rd1[...] = jnp.where(in0, dum_d, dstv - NHALF)


def _build_indices(src, dst, et):
    BE = 4000
    grid = (E // BE,)
    spec = pl.BlockSpec((BE, 1), lambda i: (i, 0))
    outs = pl.pallas_call(
        _idx_body,
        grid=grid,
        in_specs=[spec, spec, spec],
        out_specs=[spec] * 6,
        out_shape=[jax.ShapeDtypeStruct((E, 1), jnp.int32)] * 6,
    )(src.reshape(E, 1), dst.reshape(E, 1), et.reshape(E, 1))
    return tuple(o.reshape(E) for o in outs)


def _pre_body(nfa, nfi, wsrc, wdst, atta, attb, ga, gb):
    nf = jnp.concatenate([nfa[...], nfi[...]], axis=1)
    att_a = jnp.dot(nfi[...], atta[...], preferred_element_type=jnp.float32)
    att_b = jnp.dot(nfi[...], attb[...], preferred_element_type=jnp.float32)
    for t in range(T):
        ga[t, :, 0:H] = jnp.dot(nf, wsrc[t], preferred_element_type=jnp.float32)
        ga[t, :, H:2 * H] = att_a
        gb[t, :, 0:H] = jnp.dot(nf, wdst[t], preferred_element_type=jnp.float32)
        gb[t, :, H:2 * H] = att_b


def _precompute(nfa, nfi, wsrc, wdst, atta, attb):
    BN = 2000
    grid = (N // BN,)
    full = lambda s: pl.BlockSpec(s, lambda i: tuple(0 for _ in s))
    return pl.pallas_call(
        _pre_body,
        grid=grid,
        in_specs=[
            pl.BlockSpec((BN, D), lambda i: (i, 0)),
            pl.BlockSpec((BN, D), lambda i: (i, 0)),
            full((T, 2 * D, H)),
            full((T, 2 * D, H)),
            full((D, H)),
            full((D, H)),
        ],
        out_specs=[
            pl.BlockSpec((T, BN, 2 * H), lambda i: (0, i, 0)),
            pl.BlockSpec((T, BN, 2 * H), lambda i: (0, i, 0)),
        ],
        out_shape=[
            jax.ShapeDtypeStruct((T, N, 2 * H), jnp.float32),
            jax.ShapeDtypeStruct((T, N, 2 * H), jnp.float32),
        ],
    )(nfa, nfi, wsrc, wdst, atta, attb)


def _gather_body(ga_hbm, gb_hbm, rowsa_hbm, rowsb_hbm, rowsd_hbm, zd_hbm,
                 ones_hbm, outa_hbm, outb_hbm, deg_out,
                 idxa0, idxb0, dstd0, bufa0, bufb0,
                 idxa1, idxb1, dstd1, bufa1, bufb1,
                 onesb, dst16, zbuf, degtab,
                 seml0, seml1, semg0, semg1, semw0, semw1):
    c = lax.axis_index("c")
    s = lax.axis_index("s")
    wid = s * NC + c
    base = wid * EW
    dbase = c * E + base

    pltpu.sync_copy(zd_hbm, zbuf)
    for k in range(DSTRIDE // DZC):
        pltpu.sync_copy(zbuf, degtab.at[pl.ds(s * DSTRIDE + k * DZC, DZC)])
    pltpu.sync_copy(ones_hbm, onesb)
    plsc.subcore_barrier()

    sets = ((idxa0, idxb0, dstd0, bufa0, bufb0, seml0, semg0, semw0),
            (idxa1, idxb1, dstd1, bufa1, bufb1, seml1, semg1, semw1))

    def fire_loads(off, st):
        ia, ib, dd, ba, bb, sl, sg, sw = st
        pltpu.async_copy(rowsa_hbm.at[pl.ds(base + off, GCH)], ia, sl)
        pltpu.async_copy(rowsb_hbm.at[pl.ds(base + off, GCH)], ib, sl)
        pltpu.async_copy(rowsd_hbm.at[pl.ds(dbase + off, GCH)], dd, sl)

    def drain_loads(off, st):
        ia, ib, dd, ba, bb, sl, sg, sw = st
        pltpu.make_async_copy(rowsa_hbm.at[pl.ds(base + off, GCH)], ia, sl).wait()
        pltpu.make_async_copy(rowsb_hbm.at[pl.ds(base + off, GCH)], ib, sl).wait()
        pltpu.make_async_copy(rowsd_hbm.at[pl.ds(dbase + off, GCH)], dd, sl).wait()

    def drain_writes(st):
        ia, ib, dd, ba, bb, sl, sg, sw = st
        pltpu.make_async_copy(ba, outa_hbm.at[pl.ds(base, GCH)], sw).wait()
        pltpu.make_async_copy(bb, outb_hbm.at[pl.ds(base, GCH)], sw).wait()

    def sub_iter(j, st):
        ia, ib, dd, ba, bb, sl, sg, sw = st

        @pl.when(j >= 2)
        def _():
            drain_writes(st)

        drain_loads(j * GCH, st)
        cpa = pltpu.async_copy(ga_hbm.at[ia], ba, sg)
        cpb = pltpu.async_copy(gb_hbm.at[ib], bb, sg)
        pltpu.sync_copy(onesb, degtab.at[dd], add=True)
        cpa.wait()
        cpb.wait()

        @pl.when(j < NFULL - 2)
        def _():
            fire_loads((j + 2) * GCH, st)

        pltpu.async_copy(ba, outa_hbm.at[pl.ds(base + j * GCH, GCH)], sw)
        pltpu.async_copy(bb, outb_hbm.at[pl.ds(base + j * GCH, GCH)], sw)

    fire_loads(0, sets[0])
    fire_loads(GCH, sets[1])

    def pair(i, carry):
        sub_iter(2 * i, sets[0])
        sub_iter(2 * i + 1, sets[1])
        return carry

    lax.fori_loop(0, NFULL // 2, pair, 0)
    drain_writes(sets[0])
    drain_writes(sets[1])

    off = EW - GCH
    pltpu.sync_copy(rowsa_hbm.at[pl.ds(base + off, GCH)], idxa0)
    pltpu.sync_copy(rowsb_hbm.at[pl.ds(base + off, GCH)], idxb0)
    cpa = pltpu.async_copy(ga_hbm.at[idxa0], bufa0, semg0)
    cpb = pltpu.async_copy(gb_hbm.at[idxb0], bufb0, semg0)
    cpa.wait()
    cpb.wait()
    pltpu.sync_copy(bufa0, outa_hbm.at[pl.ds(base + off, GCH)])
    pltpu.sync_copy(bufb0, outb_hbm.at[pl.ds(base + off, GCH)])
    pltpu.sync_copy(rowsd_hbm.at[pl.ds(dbase + EW - GTAIL, GTAIL)], dst16)
    pltpu.sync_copy(onesb.at[pl.ds(0, GTAIL)], degtab.at[dst16], add=True)

    plsc.subcore_barrier()
    for k in range(DSTRIDE // DZC):
        pltpu.sync_copy(degtab.at[pl.ds(s * DSTRIDE + k * DZC, DZC)], zbuf)
        pltpu.sync_copy(zbuf, deg_out.at[c, pl.ds(s * DSTRIDE + k * DZC, DZC)])


def _gather(ga2, gb2, rowsa, rowsb, rowsd_cat):
    zd = jnp.zeros((DZC, H), jnp.float32)
    ones = jnp.ones((GCH, H), jnp.float32)
    mesh = plsc.VectorSubcoreMesh(core_axis_name="c", subcore_axis_name="s")
    fn = functools.partial(
        pl.kernel,
        mesh=mesh,
        out_type=[
            jax.ShapeDtypeStruct((E, 2 * H), jnp.float32),
            jax.ShapeDtypeStruct((E, 2 * H), jnp.float32),
            jax.ShapeDtypeStruct((NC, DTAB, H), jnp.float32),
        ],
        scratch_types=[
            pltpu.VMEM((GCH,), jnp.int32),
            pltpu.VMEM((GCH,), jnp.int32),
            pltpu.VMEM((GCH,), jnp.int32),
            pltpu.VMEM((GCH, 2 * H), jnp.float32),
            pltpu.VMEM((GCH, 2 * H), jnp.float32),
            pltpu.VMEM((GCH,), jnp.int32),
            pltpu.VMEM((GCH,), jnp.int32),
            pltpu.VMEM((GCH,), jnp.int32),
            pltpu.VMEM((GCH, 2 * H), jnp.float32),
            pltpu.VMEM((GCH, 2 * H), jnp.float32),
            pltpu.VMEM((GCH, H), jnp.float32),
            pltpu.VMEM((GTAIL,), jnp.int32),
            pltpu.VMEM((DZC, H), jnp.float32),
            pltpu.VMEM_SHARED((DTAB, H), jnp.float32),
            pltpu.SemaphoreType.DMA,
            pltpu.SemaphoreType.DMA,
            pltpu.SemaphoreType.DMA,
            pltpu.SemaphoreType.DMA,
            pltpu.SemaphoreType.DMA,
            pltpu.SemaphoreType.DMA,
        ],
    )(_gather_body)
    return fn(ga2, gb2, rowsa, rowsb, rowsd_cat, zd, ones)


def _edge_body(ga, gb, ef, et3, wef, rb1, w2, rb2, attef, ab1, aw2, ab2, m):
    gav = ga[...]
    gbv = gb[...]
    efv = ef[...]
    et = et3[...]
    hpre = gav[:, 0:H] + gbv[:, 0:H]
    apre = gav[:, H:2 * H] + gbv[:, H:2 * H]
    ah = jax.nn.relu(
        apre + jnp.dot(efv, attef[...], preferred_element_type=jnp.float32)
        + ab1[0, :])
    logit = jnp.dot(ah, aw2[...], preferred_element_type=jnp.float32) + ab2[0, 0]
    att = jax.nn.sigmoid(logit)
    msum = jnp.zeros((ga.shape[0], OUT), jnp.float32)
    for t in range(T):
        h = jax.nn.relu(
            hpre + jnp.dot(efv, wef[t], preferred_element_type=jnp.float32)
            + rb1[t, :])
        mt = jax.nn.relu(
            jnp.dot(h, w2[t], preferred_element_type=jnp.float32) + rb2[t, :])
        msum = msum + jnp.where(et == t, mt, 0.0)
    m[...] = att * msum


def _edge_mlp(gA, gB, ef, et3, wef, rb1, w2, rb2, attef, ab1, aw2, ab2):
    BE = 4000
    grid = (E // BE,)
    full = lambda s: pl.BlockSpec(s, lambda i: tuple(0 for _ in s))
    return pl.pallas_call(
        _edge_body,
        grid=grid,
        in_specs=[
            pl.BlockSpec((BE, 2 * H), lambda i: (i, 0)),
            pl.BlockSpec((BE, 2 * H), lambda i: (i, 0)),
            pl.BlockSpec((BE, DE), lambda i: (i, 0)),
            pl.BlockSpec((BE, 1), lambda i: (i, 0)),
            full((T, DE, H)),
            full((T, H)),
            full((T, H, OUT)),
            full((T, OUT)),
            full((DE, H)),
            full((1, H)),
            full((H, 1)),
            full((1, 1)),
        ],
        out_specs=pl.BlockSpec((BE, OUT), lambda i: (i, 0)),
        out_shape=jax.ShapeDtypeStruct((E, OUT), jnp.float32),
    )(gA, gB, ef, et3, wef, rb1, w2, rb2, attef, ab1, aw2, ab2)


def _scatter_body(m_hbm, rowsm_hbm, zm_hbm, msg_out,
                  msgtab, idxm0, mbuf0, idxm1, mbuf1,
                  seml0, seml1, sema0, sema1):
    c = lax.axis_index("c")
    s = lax.axis_index("s")
    ebase = s * ES
    ibase = c * E + ebase
    sbase = s * MSTRIDE

    pltpu.sync_copy(zm_hbm, mbuf0)

    @pl.when(s < NS - 1)
    def _():
        for k in range(29):
            pltpu.sync_copy(mbuf0, msgtab.at[pl.ds(sbase + k * SCH, SCH)])
        pltpu.sync_copy(mbuf0.at[pl.ds(0, 16)],
                        msgtab.at[pl.ds(sbase + 29 * SCH, 16)])

    @pl.when(s == NS - 1)
    def _():
        for k in range(27):
            pltpu.sync_copy(mbuf0, msgtab.at[pl.ds(sbase + k * SCH, SCH)])
        pltpu.sync_copy(mbuf0.at[pl.ds(0, 8)],
                        msgtab.at[pl.ds(sbase + 27 * SCH, 8)])

    plsc.subcore_barrier()

    sets = ((idxm0, mbuf0, seml0, sema0), (idxm1, mbuf1, seml1, sema1))

    def fire_loads(off, st):
        ix, mb, sl, sa = st
        pltpu.async_copy(rowsm_hbm.at[pl.ds(ibase + off, SCH)], ix, sl)
        pltpu.async_copy(m_hbm.at[pl.ds(ebase + off, SCH)], mb, sl)

    def drain_loads(off, st):
        ix, mb, sl, sa = st
        pltpu.make_async_copy(rowsm_hbm.at[pl.ds(ibase + off, SCH)], ix, sl).wait()
        pltpu.make_async_copy(m_hbm.at[pl.ds(ebase + off, SCH)], mb, sl).wait()

    def drain_add(st):
        ix, mb, sl, sa = st
        pltpu.make_async_copy(zm_hbm, mb, sa).wait()

    def sub_iter(j, p):
        st = sets[p]
        other = sets[1 - p]
        ix, mb, sl, sa = st
        drain_loads(j * SCH, st)
        pltpu.async_copy(mb, msgtab.at[ix], sa, add=True)

        @pl.when(j >= 1)
        def _():
            drain_add(other)

        @pl.when(j < NSCH - 1)
        def _():
            fire_loads((j + 1) * SCH, other)

    fire_loads(0, sets[0])

    def pair(i, carry):
        sub_iter(2 * i, 0)
        sub_iter(2 * i + 1, 1)
        return carry

    lax.fori_loop(0, NSCH // 2, pair, 0)
    sub_iter(NSCH - 1, 0)
    drain_add(sets[0])

    plsc.subcore_barrier()

    @pl.when(s < NS - 1)
    def _():
        for k in range(29):
            pltpu.sync_copy(msgtab.at[pl.ds(sbase + k * SCH, SCH)], mbuf0)
            pltpu.sync_copy(mbuf0, msg_out.at[c, pl.ds(sbase + k * SCH, SCH)])
        pltpu.sync_copy(msgtab.at[pl.ds(sbase + 29 * SCH, 16)],
                        mbuf0.at[pl.ds(0, 16)])
        pltpu.sync_copy(mbuf0.at[pl.ds(0, 16)],
                        msg_out.at[c, pl.ds(sbase + 29 * SCH, 16)])

    @pl.when(s == NS - 1)
    def _():
        for k in range(27):
            pltpu.sync_copy(msgtab.at[pl.ds(sbase + k * SCH, SCH)], mbuf0)
            pltpu.sync_copy(mbuf0, msg_out.at[c, pl.ds(sbase + k * SCH, SCH)])
        pltpu.sync_copy(msgtab.at[pl.ds(sbase + 27 * SCH, 8)],
                        mbuf0.at[pl.ds(0, 8)])
        pltpu.sync_copy(mbuf0.at[pl.ds(0, 8)],
                        msg_out.at[c, pl.ds(sbase + 27 * SCH, 8)])


def _scatter(m, rowsm_cat):
    zm = jnp.zeros((SCH, OUT), jnp.float32)
    mesh = plsc.VectorSubcoreMesh(core_axis_name="c", subcore_axis_name="s")
    fn = functools.partial(
        pl.kernel,
        mesh=mesh,
        out_type=jax.ShapeDtypeStruct((NC, MTAB, OUT), jnp.float32),
        scratch_types=[
            pltpu.VMEM_SHARED((MTAB, OUT), jnp.float32),
            pltpu.VMEM((SCH,), jnp.int32),
            pltpu.VMEM((SCH, OUT), jnp.float32),
            pltpu.VMEM((SCH,), jnp.int32),
            pltpu.VMEM((SCH, OUT), jnp.float32),
            pltpu.SemaphoreType.DMA,
            pltpu.SemaphoreType.DMA,
            pltpu.SemaphoreType.DMA,
            pltpu.SemaphoreType.DMA,
        ],
    )(_scatter_body)
    return fn(m, rowsm_cat, zm)


def _node_body(nfa, nfi, msg, deg, nt3, w1, b1, w2, b2, out):
    nf = jnp.concatenate([nfa[...], nfi[...]], axis=1)
    aggr = jnp.concatenate([jax.nn.relu(nf), msg[...]], axis=1)
    degsum = deg[:, 0:1]
    aggr = jnp.where(degsum > 0.0, aggr, 0.0)
    nt = nt3[...]
    outs = []
    for k in range(NT):
        h = jax.nn.relu(
            jnp.dot(aggr, w1[k], preferred_element_type=jnp.float32) + b1[k, :])
        outs.append(jax.nn.relu(
            jnp.dot(h, w2[k], preferred_element_type=jnp.float32) + b2[k, :]))
    out[...] = jnp.where(nt == 0, outs[0], outs[1])


def _node_update(nfa, nfi, msg, deg2, nt3, w1, b1, w2, b2):
    BN = 2000
    grid = (N // BN,)
    AGG = 2 * D + T * OUT
    full = lambda s: pl.BlockSpec(s, lambda i: tuple(0 for _ in s))
    return pl.pallas_call(
        _node_body,
        grid=grid,
        in_specs=[
            pl.BlockSpec((BN, D), lambda i: (i, 0)),
            pl.BlockSpec((BN, D), lambda i: (i, 0)),
            pl.BlockSpec((BN, T * OUT), lambda i: (i, 0)),
            pl.BlockSpec((BN, H), lambda i: (i, 0)),
            pl.BlockSpec((BN, 1), lambda i: (i, 0)),
            full((NT, AGG, H)),
            full((NT, H)),
            full((NT, H, OUT)),
            full((NT, OUT)),
        ],
        out_specs=pl.BlockSpec((BN, OUT), lambda i: (i, 0)),
        out_shape=jax.ShapeDtypeStruct((N, OUT), jnp.float32),
    )(nfa, nfi, msg, deg2, nt3, w1, b1, w2, b2)


def kernel(node_feature, nf_init, ef_init, edge_index, e_type, node_type,
           rel_W1, rel_b1, rel_W2, rel_b2,
           att_W1, att_b1, att_W2, att_b2,
           upd_W1, upd_b1, upd_W2, upd_b2):
    src = edge_index[0]
    dst = edge_index[1]

    wsrc = rel_W1[:, :2 * D, :]
    wdst = rel_W1[:, 2 * D:4 * D, :]
    wef = rel_W1[:, 4 * D:, :]
    atta = att_W1[:D, :]
    attb = att_W1[D:2 * D, :]
    attef = att_W1[2 * D:, :]

    rowsa, rowsb, rm0, rm1, rd0, rd1 = _build_indices(src, dst, e_type)
    GA, GB = _precompute(node_feature, nf_init, wsrc, wdst, atta, attb)
    gA, gB, deg_raw = _gather(GA.reshape(T * N, 2 * H),
                              GB.reshape(T * N, 2 * H), rowsa, rowsb,
                              jnp.concatenate([rd0, rd1]))

    et3 = e_type.reshape(E, 1)
    m = _edge_mlp(gA, gB, ef_init, et3, wef, rel_b1, rel_W2, rel_b2,
                  attef, att_b1.reshape(1, H), att_W2, att_b2.reshape(1, 1))

    msg_raw = _scatter(m, jnp.concatenate([rm0, rm1]))
    msg = msg_raw[:, :MROWS, :].reshape(NC, NHALF, T * OUT).reshape(N, T * OUT)
    deg2 = deg_raw[:, :NHALF, :].reshape(N, H)

    nt3 = node_type.reshape(N, 1)
    return _node_update(node_feature, nf_init, msg, deg2, nt3,
                        upd_W1, upd_b1, upd_W2, upd_b2)
